# baseline (speedup 1.0000x reference)
import functools

import jax
import jax.numpy as jnp
from jax import lax
from jax.experimental import pallas as pl
from jax.experimental.pallas import tpu as pltpu
from jax.experimental.pallas import tpu_sc as plsc

EPS_BN = 1e-5
_NC, _NS = 2, 16
_NW = _NC * _NS


@functools.lru_cache(maxsize=None)
def _sc_gather(rows_table, n_out, d):
    per_sub = n_out // _NW
    assert n_out % _NW == 0
    r = 128
    while r > 8 and r * d * 4 * 2 > 760_000 // 2:
        r //= 2
    r = min(r, per_sub)
    iters = per_sub // r
    assert per_sub % r == 0, (per_sub, r)
    mesh = plsc.VectorSubcoreMesh(core_axis_name="c", subcore_axis_name="s")

    @functools.partial(
        pl.kernel,
        mesh=mesh,
        compiler_params=pltpu.CompilerParams(use_tc_tiling_on_sc=False),
        out_type=jax.ShapeDtypeStruct((n_out, d), jnp.float32),
        scratch_types=[
            pltpu.VMEM((2, r), jnp.int32),
            pltpu.VMEM((2, r, d), jnp.float32),
            pltpu.SemaphoreType.DMA,
            pltpu.SemaphoreType.DMA,
        ],
    )
    def gather_k(table_hbm, idx_hbm, out_hbm, idx_v, rows_v, sem0, sem1):
        wid = lax.axis_index("s") * _NC + lax.axis_index("c")
        base = wid * per_sub
        sems = (sem0, sem1)

        def load_idx(j, c):
            pltpu.sync_copy(idx_hbm.at[pl.ds(base + j * r, r)], idx_v.at[c])

        def fire(c):
            return pltpu.async_copy(table_hbm.at[idx_v.at[c]], rows_v.at[c], sems[c])

        def writeback(j, c):
            pltpu.sync_copy(rows_v.at[c], out_hbm.at[pl.ds(base + j * r, r)])

        if iters == 1:
            load_idx(0, 0)
            fire(0).wait()
            writeback(0, 0)
        else:
            load_idx(0, 0)
            fire(0)

            def pair(p, carry):
                j0 = p * 2

                def half(j, c):
                    load_idx(j + 1, 1 - c)
                    pltpu.make_async_copy(table_hbm.at[idx_v.at[c]], rows_v.at[c], sems[c]).wait()
                    fire(1 - c)
                    writeback(j, c)

                half(j0, 0)
                half(j0 + 1, 1)
                return carry

            lax.fori_loop(0, iters // 2 - 1, pair, 0)
            je = iters - 2
            load_idx(je + 1, 1)
            pltpu.make_async_copy(table_hbm.at[idx_v.at[0]], rows_v.at[0], sems[0]).wait()
            fire(1)
            writeback(je, 0)
            pltpu.make_async_copy(table_hbm.at[idx_v.at[1]], rows_v.at[1], sems[1]).wait()
            writeback(je + 1, 1)

    return gather_k


def _gather_rows(table, idx):
    B, N, D = table.shape
    _, P, K = idx.shape
    idxf = (idx.astype(jnp.int32) + (jnp.arange(B, dtype=jnp.int32) * N)[:, None, None])
    g = _sc_gather(B * N, B * P * K, D)(table.reshape(B * N, D), idxf.reshape(-1))
    return g.reshape(B, P, K, D)


def _pad16(c):
    return (c + 15) // 16 * 16


def _build_table(parts):
    t = jnp.concatenate(parts, axis=-1)
    d = _pad16(t.shape[-1])
    if d != t.shape[-1]:
        t = jnp.pad(t, ((0, 0), (0, 0), (0, d - t.shape[-1])))
    return t


def _bn_rows(p, x):
    scale = p['g'] / jnp.sqrt(p['rv'] + EPS_BN)
    return (x - p['rm']) * scale + p['b']


def _conv_rows(p, x):
    return x @ p['w'].T + p['b']


def _fka_tc_body(scal_ref, g_ref, sup_ref, fc1t_ref, a2_ref, b2_ref, a3_ref, b3_ref, cvr_ref, bias_ref,
                 out_ref, *, c_feat, pos_off, K):
    TP = g_ref.shape[1]
    S = 16
    alpha = scal_ref[0]
    beta = scal_ref[1]
    inv_nr = scal_ref[2]
    g = g_ref[0]
    xg = g[..., :c_feat]
    pts = g[..., pos_off:pos_off + 3] - sup_ref[0][:, None, :]
    dist = jnp.sqrt(jnp.sum(pts * pts, axis=-1))
    dw = jax.nn.sigmoid(-alpha * dist + beta)
    dws = jnp.sum(dw, axis=-1, keepdims=True)
    dws = dws + (dws == 0).astype(dw.dtype) + 1e-6
    dw = dw / dws * K
    dwe = dw[:, :, None]

    pts2 = (pts * inv_nr).reshape(TP * K, 3)
    mat = jnp.maximum(jnp.dot(pts2, fc1t_ref[...], preferred_element_type=jnp.float32), 0.0)
    mat3 = mat.reshape(TP, K, S)
    mp1 = jnp.max(mat3 * dwe, axis=1)
    mat = jnp.dot(mat, a2_ref[...], preferred_element_type=jnp.float32)
    mp1c = jnp.dot(mp1, b2_ref[...], preferred_element_type=jnp.float32)
    mat3 = jnp.maximum(mat.reshape(TP, K, S) + mp1c[:, None, :], 0.0)
    mp2 = jnp.max(mat3 * dwe, axis=1)
    mat = jnp.dot(mat3.reshape(TP * K, S), a3_ref[...], preferred_element_type=jnp.float32)
    mp2c = jnp.dot(mp2, b3_ref[...], preferred_element_type=jnp.float32)
    mat3 = jnp.maximum(mat.reshape(TP, K, S) + mp2c[:, None, :], 0.0) * dwe

    cols = []
    for s in range(S):
        ms = mat3[:, :, s:s + 1]
        cols.append(jnp.sum(xg * ms, axis=1))
    aligned = jnp.concatenate(cols, axis=-1)
    out = jnp.dot(aligned, cvr_ref[...], preferred_element_type=jnp.float32) + bias_ref[...][None, :]
    out_ref[0] = out


def _fka_math(p, g, c_feat, pos_off, sup_rows):
    B, P, K, D = g.shape
    S = 16
    O = p['cv'].shape[0]
    tp = min(512, P)
    fc1t = p['fc1'].T
    a2 = p['fc2'][:, :S].T
    b2 = p['fc2'][:, S:].T
    a3 = p['fc3'][:, :S].T
    b3 = p['fc3'][:, S:].T
    cvr = jnp.transpose(p['cv'], (2, 1, 0)).reshape(S * c_feat, O)
    bias = p['cv_b'] if 'cv_b' in p else jnp.zeros((O,), jnp.float32)
    scal = jnp.stack([p['alpha'], p['beta'], 1.0 / p['norm_radius']])
    return pl.pallas_call(
        functools.partial(_fka_tc_body, c_feat=c_feat, pos_off=pos_off, K=K),
        grid=(B, P // tp),
        in_specs=[
            pl.BlockSpec(memory_space=pltpu.SMEM),
            pl.BlockSpec((1, tp, K, D), lambda b, t: (b, t, 0, 0)),
            pl.BlockSpec((1, tp, 3), lambda b, t: (b, t, 0)),
            pl.BlockSpec((3, S), lambda b, t: (0, 0)),
            pl.BlockSpec((S, S), lambda b, t: (0, 0)),
            pl.BlockSpec((S, S), lambda b, t: (0, 0)),
            pl.BlockSpec((S, S), lambda b, t: (0, 0)),
            pl.BlockSpec((S, S), lambda b, t: (0, 0)),
            pl.BlockSpec((S * c_feat, O), lambda b, t: (0, 0)),
            pl.BlockSpec((O,), lambda b, t: (0,)),
        ],
        out_specs=pl.BlockSpec((1, tp, O), lambda b, t: (b, t, 0)),
        out_shape=jax.ShapeDtypeStruct((B, P, O), jnp.float32),
    )(scal, g, sup_rows, fc1t, a2, b2, a3, b3, cvr, bias)


def _res_block(p, xr, pos_rows, sup_rows, idx):
    B, N, cin = xr.shape
    P = idx.shape[1]
    down = (N != P)
    h = jax.nn.relu(_bn_rows(p['bn0'], _conv_rows(p['cv0'], xr)))
    c2 = h.shape[-1]
    parts = [h, xr, pos_rows] if down else [h, pos_rows]
    g = _gather_rows(_build_table(parts), idx)
    pos_off = c2 + (cin if down else 0)
    f = _fka_math(p['cv1'], g, c2, pos_off, sup_rows)
    f = jax.nn.relu(_bn_rows(p['bn1'], f))
    f = _bn_rows(p['bn2'], _conv_rows(p['cv2'], f))
    xs = jnp.max(g[..., c2:c2 + cin], axis=2) if down else xr
    if 'shortcut' in p:
        xs = _conv_rows(p['shortcut'], xs)
    return jax.nn.relu(f + xs)


def _head_kernel(x4_ref, w_ref, b_ref, out_ref):
    xm = jnp.mean(x4_ref[...], axis=1)
    out_ref[...] = jnp.dot(xm, w_ref[...].T, preferred_element_type=jnp.float32) + b_ref[...][None, :]


def kernel(x, pos, support1, support2, support3, support4, ids0, ids10, ids11, ids20, ids21, ids30, ids31, ids40, ids41, params):
    xr = x.transpose(0, 2, 1)
    pr = pos.transpose(0, 2, 1)
    s1r = support1.transpose(0, 2, 1)
    s2r = support2.transpose(0, 2, 1)
    s3r = support3.transpose(0, 2, 1)
    s4r = support4.transpose(0, 2, 1)

    g0 = _gather_rows(_build_table([xr, pr]), ids0)
    x0 = _fka_math(params['cv0'], g0, 3, 3, pr)
    x0 = jax.nn.relu(_bn_rows(params['bn0'], x0))
    x0 = _res_block(params['b01'], x0, pr, pr, ids0)
    x1 = _res_block(params['b10'], x0, pr, s1r, ids10)
    x1 = _res_block(params['b11'], x1, s1r, s1r, ids11)
    x2 = _res_block(params['b20'], x1, s1r, s2r, ids20)
    x2 = _res_block(params['b21'], x2, s2r, s2r, ids21)
    x3 = _res_block(params['b30'], x2, s2r, s3r, ids30)
    x3 = _res_block(params['b31'], x3, s3r, s3r, ids31)
    x4 = _res_block(params['b40'], x3, s3r, s4r, ids40)
    x4 = _res_block(params['b41'], x4, s4r, s4r, ids41)

    w = params['fcout']['w']
    return pl.pallas_call(
        _head_kernel,
        out_shape=jax.ShapeDtypeStruct((x4.shape[0], w.shape[0]), jnp.float32),
    )(x4, w, params['fcout']['b'])


# speedup vs baseline: 2.6797x; 2.6797x over previous
import functools

import jax
import jax.numpy as jnp
from jax import lax
from jax.experimental import pallas as pl
from jax.experimental.pallas import tpu as pltpu
from jax.experimental.pallas import tpu_sc as plsc

EPS_BN = 1e-5
_NC, _NS = 2, 16
_NW = _NC * _NS


@functools.lru_cache(maxsize=None)
def _sc_gather(rows_table, n_out, d):
    per_sub = n_out // _NW
    assert n_out % _NW == 0
    r = 128
    while r > 8 and r * d * 4 * 2 > 760_000 // 2:
        r //= 2
    r = min(r, per_sub)
    iters = per_sub // r
    assert per_sub % r == 0, (per_sub, r)
    mesh = plsc.VectorSubcoreMesh(core_axis_name="c", subcore_axis_name="s")

    @functools.partial(
        pl.kernel,
        mesh=mesh,
        compiler_params=pltpu.CompilerParams(use_tc_tiling_on_sc=False),
        out_type=jax.ShapeDtypeStruct((n_out, d), jnp.float32),
        scratch_types=[
            pltpu.VMEM((2, r), jnp.int32),
            pltpu.VMEM((2, r, d), jnp.float32),
            pltpu.SemaphoreType.DMA,
            pltpu.SemaphoreType.DMA,
        ],
    )
    def gather_k(table_hbm, idx_hbm, out_hbm, idx_v, rows_v, sem0, sem1):
        wid = lax.axis_index("s") * _NC + lax.axis_index("c")
        base = wid * per_sub
        sems = (sem0, sem1)

        def load_idx(j, c):
            pltpu.sync_copy(idx_hbm.at[pl.ds(base + j * r, r)], idx_v.at[c])

        def fire(c):
            return pltpu.async_copy(table_hbm.at[idx_v.at[c]], rows_v.at[c], sems[c])

        def writeback(j, c):
            pltpu.sync_copy(rows_v.at[c], out_hbm.at[pl.ds(base + j * r, r)])

        if iters == 1:
            load_idx(0, 0)
            fire(0).wait()
            writeback(0, 0)
        else:
            load_idx(0, 0)
            fire(0)

            def pair(p, carry):
                j0 = p * 2

                def half(j, c):
                    load_idx(j + 1, 1 - c)
                    pltpu.make_async_copy(table_hbm.at[idx_v.at[c]], rows_v.at[c], sems[c]).wait()
                    fire(1 - c)
                    writeback(j, c)

                half(j0, 0)
                half(j0 + 1, 1)
                return carry

            lax.fori_loop(0, iters // 2 - 1, pair, 0)
            je = iters - 2
            load_idx(je + 1, 1)
            pltpu.make_async_copy(table_hbm.at[idx_v.at[0]], rows_v.at[0], sems[0]).wait()
            fire(1)
            writeback(je, 0)
            pltpu.make_async_copy(table_hbm.at[idx_v.at[1]], rows_v.at[1], sems[1]).wait()
            writeback(je + 1, 1)

    return gather_k


def _gather_rows(table_cf, idx):
    B, D, N = table_cf.shape
    _, P, K = idx.shape
    table = table_cf.transpose(0, 2, 1).reshape(B * N, D)
    idxt = idx.transpose(0, 2, 1)
    idxf = (idxt.astype(jnp.int32) + (jnp.arange(B, dtype=jnp.int32) * N)[:, None, None])
    g = _sc_gather(B * N, B * P * K, D)(table, idxf.reshape(-1))
    return g.reshape(B, K, P, D)


def _pad16(c):
    return (c + 15) // 16 * 16


def _build_table(parts):
    t = jnp.concatenate(parts, axis=1)
    d = _pad16(t.shape[1])
    if d != t.shape[1]:
        t = jnp.pad(t, ((0, 0), (0, d - t.shape[1]), (0, 0)))
    return t


def _bn_fold(p):
    s = p['g'] / jnp.sqrt(p['rv'] + EPS_BN)
    return s, p['b'] - p['rm'] * s


def _bn_cf(p, x):
    s, t = _bn_fold(p)
    return x * s[None, :, None] + t[None, :, None]


def _conv_cf(p, x):
    return jnp.einsum('oc,bcn->bon', p['w'], x) + p['b'][None, :, None]


def _fka_core(scal_ref, g_ref, sup_ref, fc1_ref, a2_ref, b2_ref, a3_ref, b3_ref, cvr_ref,
              c_feat, pos_off, K):
    TP = g_ref.shape[2]
    D = g_ref.shape[3]
    S = 16
    alpha = scal_ref[0]
    beta = scal_ref[1]
    inv_nr = scal_ref[2]
    gkd = g_ref[0].reshape(K * TP, D)
    gdk = jnp.transpose(gkd, (1, 0)).reshape(D, K, TP)
    xg = gdk[:c_feat]
    pts = gdk[pos_off:pos_off + 3] - sup_ref[0][:, None, :]
    dist = jnp.sqrt(pts[0] * pts[0] + pts[1] * pts[1] + pts[2] * pts[2])
    dw = jax.nn.sigmoid(-alpha * dist + beta)
    dws = jnp.sum(dw, axis=0, keepdims=True)
    dws = dws + (dws == 0).astype(dw.dtype) + 1e-6
    dw = dw / dws * K

    pts2 = (pts * inv_nr).reshape(3, K * TP)
    mat = jnp.maximum(jnp.dot(fc1_ref[...], pts2, preferred_element_type=jnp.float32), 0.0)
    mat3 = mat.reshape(S, K, TP)
    mp1 = jnp.max(mat3 * dw[None], axis=1)
    mat = jnp.dot(a2_ref[...], mat, preferred_element_type=jnp.float32)
    mp1c = jnp.dot(b2_ref[...], mp1, preferred_element_type=jnp.float32)
    mat3 = jnp.maximum(mat.reshape(S, K, TP) + mp1c[:, None, :], 0.0)
    mp2 = jnp.max(mat3 * dw[None], axis=1)
    mat = jnp.dot(a3_ref[...], mat3.reshape(S, K * TP), preferred_element_type=jnp.float32)
    mp2c = jnp.dot(b3_ref[...], mp2, preferred_element_type=jnp.float32)
    mat3 = jnp.maximum(mat.reshape(S, K, TP) + mp2c[:, None, :], 0.0) * dw[None]

    cols = []
    for s in range(S):
        cols.append(jnp.sum(xg * mat3[s][None], axis=1))
    aligned = jnp.concatenate(cols, axis=0)
    return jnp.dot(cvr_ref[...], aligned, preferred_element_type=jnp.float32)


def _block_body_down(scal_ref, g_ref, sup_ref, fc1_ref, a2_ref, b2_ref, a3_ref, b3_ref,
                     cvr_ref, s1_ref, t1_ref, w2_ref, b2v_ref, ws_ref, bsv_ref,
                     out_ref, *, c_feat, c_in, pos_off, K):
    fka = _fka_core(scal_ref, g_ref, sup_ref, fc1_ref, a2_ref, b2_ref, a3_ref, b3_ref,
                    cvr_ref, c_feat, pos_off, K)
    f = jnp.maximum(fka * s1_ref[...][:, None] + t1_ref[...][:, None], 0.0)
    f2 = jnp.dot(w2_ref[...], f, preferred_element_type=jnp.float32) + b2v_ref[...][:, None]
    TP = g_ref.shape[2]
    gkd = g_ref[0].reshape(K * TP, g_ref.shape[3])
    gdk = jnp.transpose(gkd, (1, 0)).reshape(g_ref.shape[3], K, TP)
    xs = jnp.max(gdk[c_feat:c_feat + c_in], axis=1)
    xs = jnp.dot(ws_ref[...], xs, preferred_element_type=jnp.float32) + bsv_ref[...][:, None]
    out_ref[0] = jnp.maximum(f2 + xs, 0.0)


def _block_body_same(scal_ref, g_ref, sup_ref, xr_ref, fc1_ref, a2_ref, b2_ref, a3_ref, b3_ref,
                     cvr_ref, s1_ref, t1_ref, w2_ref, b2v_ref,
                     out_ref, *, c_feat, pos_off, K):
    fka = _fka_core(scal_ref, g_ref, sup_ref, fc1_ref, a2_ref, b2_ref, a3_ref, b3_ref,
                    cvr_ref, c_feat, pos_off, K)
    f = jnp.maximum(fka * s1_ref[...][:, None] + t1_ref[...][:, None], 0.0)
    f2 = jnp.dot(w2_ref[...], f, preferred_element_type=jnp.float32) + b2v_ref[...][:, None]
    out_ref[0] = jnp.maximum(f2 + xr_ref[0], 0.0)


def _block_body_head(scal_ref, g_ref, sup_ref, fc1_ref, a2_ref, b2_ref, a3_ref, b3_ref,
                     cvr_ref, s1_ref, t1_ref, out_ref, *, c_feat, pos_off, K):
    fka = _fka_core(scal_ref, g_ref, sup_ref, fc1_ref, a2_ref, b2_ref, a3_ref, b3_ref,
                    cvr_ref, c_feat, pos_off, K)
    out_ref[0] = jnp.maximum(fka * s1_ref[...][:, None] + t1_ref[...][:, None], 0.0)


def _fka_weights(fk, c_feat):
    S = 16
    O = fk['cv'].shape[0]
    return (
        fk['fc1'],
        fk['fc2'][:, :S], fk['fc2'][:, S:],
        fk['fc3'][:, :S], fk['fc3'][:, S:],
        jnp.transpose(fk['cv'], (0, 2, 1)).reshape(O, S * c_feat),
        jnp.stack([fk['alpha'], fk['beta'], 1.0 / fk['norm_radius']]),
    )


def _wspec(shape):
    return pl.BlockSpec(shape, lambda b, t: (0,) * len(shape))


def _common_specs(tp, K, D, S, c_feat, O):
    return [
        pl.BlockSpec(memory_space=pltpu.SMEM),
        pl.BlockSpec((1, K, tp, D), lambda b, t: (b, 0, t, 0)),
        pl.BlockSpec((1, 3, tp), lambda b, t: (b, 0, t)),
    ], [
        _wspec((S, 3)), _wspec((S, S)), _wspec((S, S)), _wspec((S, S)), _wspec((S, S)),
        _wspec((O, S * c_feat)),
    ]


def _block_tc(p, g, c_feat, c_in, pos_off, sup, xr):
    B, K, P, D = g.shape
    S = 16
    fk = p['cv1']
    O = fk['cv'].shape[0]
    cout = p['cv2']['w'].shape[0]
    tp = min(512, P)
    fc1, a2, b2, a3, b3, cvr, scal = _fka_weights(fk, c_feat)
    s1, t1 = _bn_fold(p['bn1'])
    if 'cv_b' in fk:
        t1 = t1 + fk['cv_b'] * s1
    s2, t2 = _bn_fold(p['bn2'])
    w2 = p['cv2']['w'] * s2[:, None]
    b2v = p['cv2']['b'] * s2 + t2
    head_specs, w_specs = _common_specs(tp, K, D, S, c_feat, O)
    w_specs = w_specs + [_wspec((O,)), _wspec((O,)), _wspec((cout, O)), _wspec((cout,))]
    if xr is None:
        return pl.pallas_call(
            functools.partial(_block_body_down, c_feat=c_feat, c_in=c_in, pos_off=pos_off, K=K),
            grid=(B, P // tp),
            in_specs=head_specs + w_specs + [_wspec((cout, c_in)), _wspec((cout,))],
            out_specs=pl.BlockSpec((1, cout, tp), lambda b, t: (b, 0, t)),
            out_shape=jax.ShapeDtypeStruct((B, cout, P), jnp.float32),
        )(scal, g, sup, fc1, a2, b2, a3, b3, cvr, s1, t1, w2, b2v,
          p['shortcut']['w'], p['shortcut']['b'])
    return pl.pallas_call(
        functools.partial(_block_body_same, c_feat=c_feat, pos_off=pos_off, K=K),
        grid=(B, P // tp),
        in_specs=head_specs + [pl.BlockSpec((1, c_in, tp), lambda b, t: (b, 0, t))] + w_specs,
        out_specs=pl.BlockSpec((1, cout, tp), lambda b, t: (b, 0, t)),
        out_shape=jax.ShapeDtypeStruct((B, cout, P), jnp.float32),
    )(scal, g, sup, xr, fc1, a2, b2, a3, b3, cvr, s1, t1, w2, b2v)


def _res_block(p, x, pos, sup, idx):
    B, cin, N = x.shape
    P = idx.shape[1]
    down = (N != P)
    h = jax.nn.relu(_bn_cf(p['bn0'], _conv_cf(p['cv0'], x)))
    c2 = h.shape[1]
    parts = [h, x, pos] if down else [h, pos]
    g = _gather_rows(_build_table(parts), idx)
    pos_off = c2 + (cin if down else 0)
    return _block_tc(p, g, c2, cin, pos_off, sup, None if down else x)


def _first_layer(p0, bn0, g, pos):
    B, K, P, D = g.shape
    S = 16
    c_feat = 3
    O = p0['cv'].shape[0]
    tp = min(512, P)
    fc1, a2, b2, a3, b3, cvr, scal = _fka_weights(p0, c_feat)
    s1, t1 = _bn_fold(bn0)
    t1 = t1 + p0['cv_b'] * s1
    head_specs, w_specs = _common_specs(tp, K, D, S, c_feat, O)
    return pl.pallas_call(
        functools.partial(_block_body_head, c_feat=c_feat, pos_off=3, K=K),
        grid=(B, P // tp),
        in_specs=head_specs + w_specs + [_wspec((O,)), _wspec((O,))],
        out_specs=pl.BlockSpec((1, O, tp), lambda b, t: (b, 0, t)),
        out_shape=jax.ShapeDtypeStruct((B, O, P), jnp.float32),
    )(scal, g, pos, fc1, a2, b2, a3, b3, cvr, s1, t1)


def _head_kernel(x4_ref, w_ref, b_ref, out_ref):
    xm = jnp.mean(x4_ref[...], axis=2)
    out_ref[...] = jnp.dot(xm, w_ref[...].T, preferred_element_type=jnp.float32) + b_ref[...][None, :]


def kernel(x, pos, support1, support2, support3, support4, ids0, ids10, ids11, ids20, ids21, ids30, ids31, ids40, ids41, params):
    g0 = _gather_rows(_build_table([x, pos]), ids0)
    x0 = _first_layer(params['cv0'], params['bn0'], g0, pos)
    x0 = _res_block(params['b01'], x0, pos, pos, ids0)
    x1 = _res_block(params['b10'], x0, pos, support1, ids10)
    x1 = _res_block(params['b11'], x1, support1, support1, ids11)
    x2 = _res_block(params['b20'], x1, support1, support2, ids20)
    x2 = _res_block(params['b21'], x2, support2, support2, ids21)
    x3 = _res_block(params['b30'], x2, support2, support3, ids30)
    x3 = _res_block(params['b31'], x3, support3, support3, ids31)
    x4 = _res_block(params['b40'], x3, support3, support4, ids40)
    x4 = _res_block(params['b41'], x4, support4, support4, ids41)

    w = params['fcout']['w']
    return pl.pallas_call(
        _head_kernel,
        out_shape=jax.ShapeDtypeStruct((x4.shape[0], w.shape[0]), jnp.float32),
    )(x4, w, params['fcout']['b'])
